# baseline (speedup 1.0000x reference)
import jax
import jax.numpy as jnp
from jax import lax
from jax.experimental import pallas as pl
from jax.experimental.pallas import tpu as pltpu
from jax.experimental.pallas import tpu_sc as plsc

F32 = jnp.float32
I32 = jnp.int32

NLEV = 8
R0 = 512
F = 16
NPTS = 262144
TPP = 349520
NC, NS = 2, 16
NW = NC * NS
_OFFS = [0, 262144, 327680, 344064, 348160, 349184, 349440, 349504]


def _mesh():
    return plsc.VectorSubcoreMesh(
        core_axis_name="c", subcore_axis_name="s",
        num_cores=NC, num_subcores=NS)


def _cparams():
    return pltpu.CompilerParams(
        use_tc_tiling_on_sc=False, needs_layout_passes=False)


def _wid():
    return lax.axis_index("s") * NC + lax.axis_index("c")


def _iota16():
    return lax.iota(I32, 16)


def _ds_pair(src, s0, s1, dst, d0, width_out):
    def body(ox, _):
        a = src[s0 + 2 * ox]
        b = src[s0 + 2 * ox + 1]
        c = src[s1 + 2 * ox]
        d = src[s1 + 2 * ox + 1]
        dst[d0 + ox] = (a + b + c + d) * 0.25
        return 0
    lax.fori_loop(0, width_out, body, 0)


def _pack_pair(a, b):
    ai = plsc.bitcast(a, I32) + 0x8000
    bi = plsc.bitcast(b, I32) + 0x8000
    lo = lax.bitwise_and(lax.shift_right_logical(ai, 16),
                         jnp.full((16,), 0xFFFF, I32))
    hi = lax.bitwise_and(bi, jnp.full((16,), -65536, I32))
    return lax.bitwise_or(lo, hi)


def _emit_rows(src, s0, dst, d0, nrows, width):
    def body(j, _):
        r = j // width
        xx = j - r * width
        a = src[s0 + j]
        b = src[s0 + r * width + jnp.minimum(xx + 1, width - 1)]
        dst[d0 + j] = _pack_pair(a, b)
        return 0
    lax.fori_loop(0, nrows * width, body, 0)


def _pyr14_body(tex, tbl, l4f, in_v, l1_v, l2_v, l3_v, l4_v, pair_v):
    slab = _wid()
    for plane in range(3):
        for i in range(4):
            pltpu.sync_copy(
                tex.at[plane, pl.ds(slab * 16 * R0 + i * 2048, 2048), :], in_v)
            _emit_rows(in_v, 0, pair_v, 0, 4, 512)
            pltpu.sync_copy(
                pair_v, tbl.at[plane, pl.ds(slab * 8192 + i * 2048, 2048), :])
            for jj in range(2):
                _ds_pair(in_v, jj * 1024, jj * 1024 + 512,
                         l1_v, (i * 2 + jj) * 256, 256)
        _emit_rows(l1_v, 0, pair_v, 0, 8, 256)
        pltpu.sync_copy(
            pair_v, tbl.at[plane, pl.ds(_OFFS[1] + slab * 2048, 2048), :])
        for j in range(4):
            _ds_pair(l1_v, (2 * j) * 256, (2 * j + 1) * 256, l2_v, j * 128, 128)
        _emit_rows(l2_v, 0, pair_v, 0, 4, 128)
        pltpu.sync_copy(
            pair_v.at[pl.ds(0, 512), :],
            tbl.at[plane, pl.ds(_OFFS[2] + slab * 512, 512), :])
        for j in range(2):
            _ds_pair(l2_v, (2 * j) * 128, (2 * j + 1) * 128, l3_v, j * 64, 64)
        _emit_rows(l3_v, 0, pair_v, 0, 2, 64)
        pltpu.sync_copy(
            pair_v.at[pl.ds(0, 128), :],
            tbl.at[plane, pl.ds(_OFFS[3] + slab * 128, 128), :])
        _ds_pair(l3_v, 0, 64, l4_v, 0, 32)
        _emit_rows(l4_v, 0, pair_v, 0, 1, 32)
        pltpu.sync_copy(
            pair_v.at[pl.ds(0, 32), :],
            tbl.at[plane, pl.ds(_OFFS[4] + slab * 32, 32), :])
        pltpu.sync_copy(l4_v, l4f.at[plane, pl.ds(slab * 32, 32), :])


def _pyr14_kernel():
    return pl.kernel(
        _pyr14_body,
        out_type=(jax.ShapeDtypeStruct((3, TPP, F), I32),
                  jax.ShapeDtypeStruct((3, 1024, F), F32)),
        mesh=_mesh(),
        compiler_params=_cparams(),
        scratch_types=[pltpu.VMEM((2048, F), F32),
                       pltpu.VMEM((2048, F), F32),
                       pltpu.VMEM((512, F), F32),
                       pltpu.VMEM((128, F), F32),
                       pltpu.VMEM((32, F), F32),
                       pltpu.VMEM((2048, F), I32)],
    )


def _pyr57_body(l4f, tail, in_v, l5_v, l6_v, l7_v, pair_v):
    w = _wid()

    @pl.when(w < 3)
    def _():
        pltpu.sync_copy(l4f.at[w, :, :], in_v)

        def b5(r, _):
            _ds_pair(in_v, (2 * r) * 32, (2 * r + 1) * 32, l5_v, r * 16, 16)
            return 0
        lax.fori_loop(0, 16, b5, 0)
        _emit_rows(l5_v, 0, pair_v, 0, 16, 16)
        pltpu.sync_copy(pair_v.at[pl.ds(0, 256), :],
                        tail.at[w, pl.ds(0, 256), :])

        def b6(r, _):
            _ds_pair(l5_v, (2 * r) * 16, (2 * r + 1) * 16, l6_v, r * 8, 8)
            return 0
        lax.fori_loop(0, 8, b6, 0)
        _emit_rows(l6_v, 0, pair_v, 0, 8, 8)
        pltpu.sync_copy(pair_v.at[pl.ds(0, 64), :],
                        tail.at[w, pl.ds(256, 64), :])

        def b7(r, _):
            _ds_pair(l6_v, (2 * r) * 8, (2 * r + 1) * 8, l7_v, r * 4, 4)
            return 0
        lax.fori_loop(0, 4, b7, 0)
        _emit_rows(l7_v, 0, pair_v, 0, 4, 4)
        pltpu.sync_copy(pair_v.at[pl.ds(0, 16), :],
                        tail.at[w, pl.ds(320, 16), :])


def _pyr57_kernel():
    return pl.kernel(
        _pyr57_body,
        out_type=jax.ShapeDtypeStruct((3, 336, F), I32),
        mesh=_mesh(),
        compiler_params=_cparams(),
        scratch_types=[pltpu.VMEM((1024, F), F32),
                       pltpu.VMEM((256, F), F32),
                       pltpu.VMEM((64, F), F32),
                       pltpu.VMEM((16, F), F32),
                       pltpu.VMEM((256, F), I32)],
    )


_BC = 512
_NCH = 3 * (NPTS // NW) // _BC


def _main_body(tbl, x_hbm, lev_hbm, out_hbm,
               x_v0, x_v1, lev_v0, lev_v1, idx_v0, idx_v1, w8_v0, w8_v1,
               rows_v0, rows_v1, outst_v, semg0, semg1):
    w = _wid()
    npw = NPTS // NW
    cpp = npw // _BC
    iota = _iota16()

    def chunk_coords(t):
        p = t // cpp
        c = t - p * cpp
        return p, w * npw + c * _BC

    def load_and_fire(t, x_v, lev_v, idx_v, w8_v, rows_v, semg):
        p, n0 = chunk_coords(t)
        pltpu.sync_copy(x_hbm.at[pl.ds(3 * n0, 3 * _BC)], x_v)
        pltpu.sync_copy(lev_hbm.at[pl.ds(n0, _BC)], lev_v)
        c0 = jnp.where(p == 0, 1, 0)
        c1 = jnp.where(p == 2, 1, 2)
        prow = p * TPP

        def grp(g, _):
            pv = g * 16 + iota
            u = plsc.load_gather(x_v, [3 * pv + c0])
            v = plsc.load_gather(x_v, [3 * pv + c1])
            lv = plsc.load_gather(lev_v, [pv])
            lv = jnp.clip(lv, 0.0, float(NLEV - 1))
            l0i = jnp.clip(lv.astype(I32), 0, NLEV - 1)
            l1i = jnp.minimum(l0i + 1, NLEV - 1)
            fl = lv - l0i.astype(F32)
            for s, (li, wl) in enumerate(((l0i, 1.0 - fl), (l1i, fl))):
                resi = lax.shift_right_logical(jnp.full((16,), R0, I32), li)
                resf = resi.astype(F32)
                offs = jnp.full((16,), _OFFS[NLEV - 1], I32)
                for l in range(NLEV - 2, -1, -1):
                    offs = jnp.where(li == l, _OFFS[l], offs)
                uu = u * resf - 0.5
                vv = v * resf - 0.5
                x0i = (uu + 1.0).astype(I32) - 1
                y0i = (vv + 1.0).astype(I32) - 1
                px = jnp.clip(x0i, 0, resi - 2)
                fxp = jnp.clip(uu - px.astype(F32), 0.0, 1.0)
                iy0 = jnp.clip(y0i, 0, resi - 1)
                iy1 = jnp.clip(y0i + 1, 0, resi - 1)
                fy = vv - y0i.astype(F32)
                base = prow + offs + px
                for jy, (iy, wy) in enumerate(((iy0, 1.0 - fy), (iy1, fy))):
                    k2 = s * 2 + jy
                    idx_v[k2, pl.ds(g * 16, 16)] = base + iy * resi
                    wly = wl * wy
                    w8_v[2 * k2, pl.ds(g * 16, 16)] = wly * (1.0 - fxp)
                    w8_v[2 * k2 + 1, pl.ds(g * 16, 16)] = wly * fxp
            return 0
        lax.fori_loop(0, _BC // 16, grp, 0)
        for k2 in range(4):
            for q in range(_BC // 128):
                pltpu.async_copy(
                    tbl.at[idx_v.at[k2, pl.ds(q * 128, 128)]],
                    rows_v.at[pl.ds((k2 * (_BC // 128) + q) * 128, 128), :],
                    semg)

    def drain(rows_v, semg):
        pltpu.make_async_copy(
            tbl.at[pl.ds(0, 4 * _BC), :], rows_v, semg).wait()

    def compute_store(t, w8_v, rows_v):
        p, n0 = chunk_coords(t)
        mhi = jnp.full((16,), -65536, I32)

        def grp(g, _):
            p0 = g * 16
            wvec = [w8_v[k, pl.ds(p0, 16)] for k in range(8)]
            for i in range(16):
                lane = jnp.full((16,), i, I32)
                acc = None
                for k2 in range(4):
                    wrd = rows_v[k2 * _BC + p0 + i]
                    tl = plsc.bitcast(lax.shift_left(wrd, 16), F32)
                    tr = plsc.bitcast(lax.bitwise_and(wrd, mhi), F32)
                    a = wvec[2 * k2][lane] * tl + wvec[2 * k2 + 1][lane] * tr
                    acc = a if acc is None else acc + a
                outst_v[p0 + i] = acc
            return 0
        lax.fori_loop(0, _BC // 16, grp, 0)
        pltpu.sync_copy(outst_v,
                        out_hbm.at[pl.ds(n0, _BC), pl.ds(p * F, F)])

    set0 = (x_v0, lev_v0, idx_v0, w8_v0, rows_v0, semg0)
    set1 = (x_v1, lev_v1, idx_v1, w8_v1, rows_v1, semg1)
    load_and_fire(0, *set0)

    def body(i, _):
        t0 = 2 * i
        load_and_fire(t0 + 1, *set1)
        drain(rows_v0, semg0)
        compute_store(t0, w8_v0, rows_v0)

        @pl.when(t0 + 2 < _NCH)
        def _():
            load_and_fire(t0 + 2, *set0)
        drain(rows_v1, semg1)
        compute_store(t0 + 1, w8_v1, rows_v1)
        return 0
    lax.fori_loop(0, _NCH // 2, body, 0)


def _main_kernel():
    return pl.kernel(
        _main_body,
        out_type=jax.ShapeDtypeStruct((NPTS, 3 * F), F32),
        mesh=_mesh(),
        compiler_params=_cparams(),
        scratch_types=[pltpu.VMEM((3 * _BC,), F32),
                       pltpu.VMEM((3 * _BC,), F32),
                       pltpu.VMEM((_BC,), F32),
                       pltpu.VMEM((_BC,), F32),
                       pltpu.VMEM((4, _BC), I32),
                       pltpu.VMEM((4, _BC), I32),
                       pltpu.VMEM((8, _BC), F32),
                       pltpu.VMEM((8, _BC), F32),
                       pltpu.VMEM((4 * _BC, F), I32),
                       pltpu.VMEM((4 * _BC, F), I32),
                       pltpu.VMEM((_BC, F), F32),
                       pltpu.SemaphoreType.DMA,
                       pltpu.SemaphoreType.DMA],
    )


def kernel(x, level, texture):
    if x.shape[0] == 0:
        return jnp.zeros([0, F * 3], dtype=F32)
    x_flat = x.reshape(3 * NPTS)
    lev_flat = level.reshape(NPTS)
    tex_flat = texture.reshape(3, R0 * R0, F)
    tbl_big, l4f = _pyr14_kernel()(tex_flat)
    tail = _pyr57_kernel()(l4f)
    tbl = lax.dynamic_update_slice(tbl_big, tail, (0, _OFFS[5], 0))
    return _main_kernel()(tbl.reshape(3 * TPP, F), x_flat, lev_flat)


# speedup vs baseline: 1.1221x; 1.1221x over previous
import jax
import jax.numpy as jnp
from jax import lax
from jax.experimental import pallas as pl
from jax.experimental.pallas import tpu as pltpu
from jax.experimental.pallas import tpu_sc as plsc

F32 = jnp.float32
I32 = jnp.int32

NLEV = 8
R0 = 512
F = 16
NPTS = 262144
TPP = 349520
NC, NS = 2, 16
NW = NC * NS
_OFFS = [0, 262144, 327680, 344064, 348160, 349184, 349440, 349504]


def _mesh():
    return plsc.VectorSubcoreMesh(
        core_axis_name="c", subcore_axis_name="s",
        num_cores=NC, num_subcores=NS)


def _cparams():
    return pltpu.CompilerParams(
        use_tc_tiling_on_sc=False, needs_layout_passes=False)


def _wid():
    return lax.axis_index("s") * NC + lax.axis_index("c")


def _iota16():
    return lax.iota(I32, 16)


def _ds_pair(src, s0, s1, dst, d0, width_out):
    def body(ox, _):
        a = src[s0 + 2 * ox]
        b = src[s0 + 2 * ox + 1]
        c = src[s1 + 2 * ox]
        d = src[s1 + 2 * ox + 1]
        dst[d0 + ox] = (a + b + c + d) * 0.25
        return 0
    lax.fori_loop(0, width_out, body, 0)


def _pack_pair(a, b):
    ai = plsc.bitcast(a, I32) + 0x8000
    bi = plsc.bitcast(b, I32) + 0x8000
    lo = lax.bitwise_and(lax.shift_right_logical(ai, 16),
                         jnp.full((16,), 0xFFFF, I32))
    hi = lax.bitwise_and(bi, jnp.full((16,), -65536, I32))
    return lax.bitwise_or(lo, hi)


def _emit_rows(src, s0, dst, d0, nrows, width):
    def body(j, _):
        r = j // width
        xx = j - r * width
        a = src[s0 + j]
        b = src[s0 + r * width + jnp.minimum(xx + 1, width - 1)]
        dst[d0 + j] = _pack_pair(a, b)
        return 0
    lax.fori_loop(0, nrows * width, body, 0)


def _pyr14_body(tex, tbl, l4f, in_v, l1_v, l2_v, l3_v, l4_v, pair_v):
    slab = _wid()
    for plane in range(3):
        pbase = plane * TPP
        for i in range(4):
            for r in range(4):
                pltpu.sync_copy(
                    tex.at[plane, slab * 16 + i * 4 + r, :, :],
                    in_v.at[pl.ds(r * 512, 512), :])
            _emit_rows(in_v, 0, pair_v, 0, 4, 512)
            pltpu.sync_copy(
                pair_v,
                tbl.at[pl.ds(pbase + slab * 8192 + i * 2048, 2048), :])
            for jj in range(2):
                _ds_pair(in_v, jj * 1024, jj * 1024 + 512,
                         l1_v, (i * 2 + jj) * 256, 256)
        _emit_rows(l1_v, 0, pair_v, 0, 8, 256)
        pltpu.sync_copy(
            pair_v, tbl.at[pl.ds(pbase + _OFFS[1] + slab * 2048, 2048), :])
        for j in range(4):
            _ds_pair(l1_v, (2 * j) * 256, (2 * j + 1) * 256, l2_v, j * 128, 128)
        _emit_rows(l2_v, 0, pair_v, 0, 4, 128)
        pltpu.sync_copy(
            pair_v.at[pl.ds(0, 512), :],
            tbl.at[pl.ds(pbase + _OFFS[2] + slab * 512, 512), :])
        for j in range(2):
            _ds_pair(l2_v, (2 * j) * 128, (2 * j + 1) * 128, l3_v, j * 64, 64)
        _emit_rows(l3_v, 0, pair_v, 0, 2, 64)
        pltpu.sync_copy(
            pair_v.at[pl.ds(0, 128), :],
            tbl.at[pl.ds(pbase + _OFFS[3] + slab * 128, 128), :])
        _ds_pair(l3_v, 0, 64, l4_v, 0, 32)
        _emit_rows(l4_v, 0, pair_v, 0, 1, 32)
        pltpu.sync_copy(
            pair_v.at[pl.ds(0, 32), :],
            tbl.at[pl.ds(pbase + _OFFS[4] + slab * 32, 32), :])
        pltpu.sync_copy(l4_v, l4f.at[plane, pl.ds(slab * 32, 32), :])


def _pyr14_kernel():
    return pl.kernel(
        _pyr14_body,
        out_type=jax.ShapeDtypeStruct((3, 1024, F), F32),
        mesh=_mesh(),
        compiler_params=_cparams(),
        scratch_types=[pltpu.VMEM((2048, F), F32),
                       pltpu.VMEM((2048, F), F32),
                       pltpu.VMEM((512, F), F32),
                       pltpu.VMEM((128, F), F32),
                       pltpu.VMEM((32, F), F32),
                       pltpu.VMEM((2048, F), I32)],
    )


def _pyr57_body(l4f, tbl, in_v, l5_v, l6_v, l7_v, pair_v):
    w = _wid()

    @pl.when(w < 3)
    def _():
        pltpu.sync_copy(l4f.at[w, :, :], in_v)

        def b5(r, _):
            _ds_pair(in_v, (2 * r) * 32, (2 * r + 1) * 32, l5_v, r * 16, 16)
            return 0
        lax.fori_loop(0, 16, b5, 0)
        _emit_rows(l5_v, 0, pair_v, 0, 16, 16)
        pltpu.sync_copy(pair_v.at[pl.ds(0, 256), :],
                        tbl.at[pl.ds(w * TPP + _OFFS[5], 256), :])

        def b6(r, _):
            _ds_pair(l5_v, (2 * r) * 16, (2 * r + 1) * 16, l6_v, r * 8, 8)
            return 0
        lax.fori_loop(0, 8, b6, 0)
        _emit_rows(l6_v, 0, pair_v, 0, 8, 8)
        pltpu.sync_copy(pair_v.at[pl.ds(0, 64), :],
                        tbl.at[pl.ds(w * TPP + _OFFS[6], 64), :])

        def b7(r, _):
            _ds_pair(l6_v, (2 * r) * 8, (2 * r + 1) * 8, l7_v, r * 4, 4)
            return 0
        lax.fori_loop(0, 4, b7, 0)
        _emit_rows(l7_v, 0, pair_v, 0, 4, 4)
        pltpu.sync_copy(pair_v.at[pl.ds(0, 16), :],
                        tbl.at[pl.ds(w * TPP + _OFFS[7], 16), :])


def _pyr57_kernel():
    return pl.kernel(
        _pyr57_body,
        out_type=(),
        mesh=_mesh(),
        compiler_params=_cparams(),
        scratch_types=[pltpu.VMEM((1024, F), F32),
                       pltpu.VMEM((256, F), F32),
                       pltpu.VMEM((64, F), F32),
                       pltpu.VMEM((16, F), F32),
                       pltpu.VMEM((256, F), I32)],
    )


_BC = 512
_NCH = 3 * (NPTS // NW) // _BC


def _main_body(tbl, x_hbm, lev_hbm, out_hbm,
               x_v0, x_v1, lev_v0, lev_v1, idx_v0, idx_v1, w8_v0, w8_v1,
               rows_v0, rows_v1, outst_v, semg0, semg1):
    w = _wid()
    npw = NPTS // NW
    cpp = npw // _BC
    iota = _iota16()

    def chunk_coords(t):
        p = t // cpp
        c = t - p * cpp
        return p, w * npw + c * _BC

    def load_and_fire(t, x_v, lev_v, idx_v, w8_v, rows_v, semg):
        p, n0 = chunk_coords(t)
        pltpu.sync_copy(x_hbm.at[pl.ds(n0, _BC), :], x_v)
        pltpu.sync_copy(lev_hbm.at[pl.ds(n0, _BC), :], lev_v)
        c0 = jnp.where(p == 0, 1, 0)
        c1 = jnp.where(p == 2, 1, 2)
        prow = p * TPP

        zero16 = jnp.zeros((16,), I32)

        def grp(g, _):
            pv = g * 16 + iota
            u = plsc.load_gather(x_v, [pv, zero16 + c0])
            v = plsc.load_gather(x_v, [pv, zero16 + c1])
            lv = plsc.load_gather(lev_v, [pv, zero16])
            lv = jnp.clip(lv, 0.0, float(NLEV - 1))
            l0i = jnp.clip(lv.astype(I32), 0, NLEV - 1)
            l1i = jnp.minimum(l0i + 1, NLEV - 1)
            fl = lv - l0i.astype(F32)
            for s, (li, wl) in enumerate(((l0i, 1.0 - fl), (l1i, fl))):
                resi = lax.shift_right_logical(jnp.full((16,), R0, I32), li)
                resf = resi.astype(F32)
                offs = jnp.full((16,), _OFFS[NLEV - 1], I32)
                for l in range(NLEV - 2, -1, -1):
                    offs = jnp.where(li == l, _OFFS[l], offs)
                uu = u * resf - 0.5
                vv = v * resf - 0.5
                x0i = (uu + 1.0).astype(I32) - 1
                y0i = (vv + 1.0).astype(I32) - 1
                px = jnp.clip(x0i, 0, resi - 2)
                fxp = jnp.clip(uu - px.astype(F32), 0.0, 1.0)
                iy0 = jnp.clip(y0i, 0, resi - 1)
                iy1 = jnp.clip(y0i + 1, 0, resi - 1)
                fy = vv - y0i.astype(F32)
                base = prow + offs + px
                for jy, (iy, wy) in enumerate(((iy0, 1.0 - fy), (iy1, fy))):
                    k2 = s * 2 + jy
                    idx_v[k2, pl.ds(g * 16, 16)] = base + iy * resi
                    wly = wl * wy
                    w8_v[2 * k2, pl.ds(g * 16, 16)] = wly * (1.0 - fxp)
                    w8_v[2 * k2 + 1, pl.ds(g * 16, 16)] = wly * fxp
            return 0
        lax.fori_loop(0, _BC // 16, grp, 0)
        for k2 in range(4):
            for q in range(_BC // 128):
                pltpu.async_copy(
                    tbl.at[idx_v.at[k2, pl.ds(q * 128, 128)]],
                    rows_v.at[pl.ds((k2 * (_BC // 128) + q) * 128, 128), :],
                    semg)

    def drain(rows_v, semg):
        pltpu.make_async_copy(
            tbl.at[pl.ds(0, 4 * _BC), :], rows_v, semg).wait()

    def compute_store(t, w8_v, rows_v):
        p, n0 = chunk_coords(t)
        mhi = jnp.full((16,), -65536, I32)

        def grp(g, _):
            p0 = g * 16
            wvec = [w8_v[k, pl.ds(p0, 16)] for k in range(8)]
            for i in range(16):
                lane = jnp.full((16,), i, I32)
                acc = None
                for k2 in range(4):
                    wrd = rows_v[k2 * _BC + p0 + i]
                    tl = plsc.bitcast(lax.shift_left(wrd, 16), F32)
                    tr = plsc.bitcast(lax.bitwise_and(wrd, mhi), F32)
                    a = wvec[2 * k2][lane] * tl + wvec[2 * k2 + 1][lane] * tr
                    acc = a if acc is None else acc + a
                outst_v[p0 + i] = acc
            return 0
        lax.fori_loop(0, _BC // 16, grp, 0)
        pltpu.sync_copy(outst_v,
                        out_hbm.at[pl.ds(n0, _BC), pl.ds(p * F, F)])

    set0 = (x_v0, lev_v0, idx_v0, w8_v0, rows_v0, semg0)
    set1 = (x_v1, lev_v1, idx_v1, w8_v1, rows_v1, semg1)
    load_and_fire(0, *set0)

    def body(i, _):
        t0 = 2 * i
        load_and_fire(t0 + 1, *set1)
        drain(rows_v0, semg0)
        compute_store(t0, w8_v0, rows_v0)

        @pl.when(t0 + 2 < _NCH)
        def _():
            load_and_fire(t0 + 2, *set0)
        drain(rows_v1, semg1)
        compute_store(t0 + 1, w8_v1, rows_v1)
        return 0
    lax.fori_loop(0, _NCH // 2, body, 0)


def _main_kernel():
    return pl.kernel(
        _main_body,
        out_type=jax.ShapeDtypeStruct((NPTS, 3 * F), F32),
        mesh=_mesh(),
        compiler_params=_cparams(),
        scratch_types=[pltpu.VMEM((_BC, 3), F32),
                       pltpu.VMEM((_BC, 3), F32),
                       pltpu.VMEM((_BC, 1), F32),
                       pltpu.VMEM((_BC, 1), F32),
                       pltpu.VMEM((4, _BC), I32),
                       pltpu.VMEM((4, _BC), I32),
                       pltpu.VMEM((8, _BC), F32),
                       pltpu.VMEM((8, _BC), F32),
                       pltpu.VMEM((4 * _BC, F), I32),
                       pltpu.VMEM((4 * _BC, F), I32),
                       pltpu.VMEM((_BC, F), F32),
                       pltpu.SemaphoreType.DMA,
                       pltpu.SemaphoreType.DMA],
    )


def kernel(x, level, texture):
    if x.shape[0] == 0:
        return jnp.zeros([0, F * 3], dtype=F32)
    tbl_ref = jax.new_ref(jnp.zeros((3 * TPP, F), I32))
    l4f = _pyr14_kernel()(texture, tbl_ref)
    _pyr57_kernel()(l4f, tbl_ref)
    return _main_kernel()(tbl_ref, x, level)


# speedup vs baseline: 1.2914x; 1.1509x over previous
import jax
import jax.numpy as jnp
from jax import lax
from jax.experimental import pallas as pl
from jax.experimental.pallas import tpu as pltpu
from jax.experimental.pallas import tpu_sc as plsc

F32 = jnp.float32
I32 = jnp.int32

NLEV = 8
R0 = 512
F = 16
NPTS = 262144
TPP = 349520
NC, NS = 2, 16
NW = NC * NS
_OFFS = [0, 262144, 327680, 344064, 348160, 349184, 349440, 349504]


def _mesh():
    return plsc.VectorSubcoreMesh(
        core_axis_name="c", subcore_axis_name="s",
        num_cores=NC, num_subcores=NS)


def _cparams():
    return pltpu.CompilerParams(
        use_tc_tiling_on_sc=False, needs_layout_passes=False)


def _wid():
    return lax.axis_index("s") * NC + lax.axis_index("c")


def _iota16():
    return lax.iota(I32, 16)


def _ds_pair(src, s0, s1, dst, d0, width_out):
    def body(ox, _):
        a = src[s0 + 2 * ox]
        b = src[s0 + 2 * ox + 1]
        c = src[s1 + 2 * ox]
        d = src[s1 + 2 * ox + 1]
        dst[d0 + ox] = (a + b + c + d) * 0.25
        return 0
    lax.fori_loop(0, width_out, body, 0)


def _pack_pair(a, b):
    ai = plsc.bitcast(a, I32) + 0x8000
    bi = plsc.bitcast(b, I32) + 0x8000
    lo = lax.bitwise_and(lax.shift_right_logical(ai, 16),
                         jnp.full((16,), 0xFFFF, I32))
    hi = lax.bitwise_and(bi, jnp.full((16,), -65536, I32))
    return lax.bitwise_or(lo, hi)


def _emit_rows(src, s0, dst, d0, nrows, width):
    def body(j, _):
        r = j // width
        xx = j - r * width
        a = src[s0 + j]
        b = src[s0 + r * width + jnp.minimum(xx + 1, width - 1)]
        dst[d0 + j] = _pack_pair(a, b)
        return 0
    lax.fori_loop(0, nrows * width, body, 0)


def _pyr14_body(tex, tbl, l4f, in_v, l1_v, l2_v, l3_v, l4_v, pair_v):
    slab = _wid()
    for plane in range(3):
        pbase = plane * TPP
        for i in range(4):
            for r in range(4):
                pltpu.sync_copy(
                    tex.at[plane, slab * 16 + i * 4 + r, :, :],
                    in_v.at[pl.ds(r * 512, 512), :])
            _emit_rows(in_v, 0, pair_v, 0, 4, 512)
            pltpu.sync_copy(
                pair_v,
                tbl.at[pl.ds(pbase + slab * 8192 + i * 2048, 2048), :])
            for jj in range(2):
                _ds_pair(in_v, jj * 1024, jj * 1024 + 512,
                         l1_v, (i * 2 + jj) * 256, 256)
        _emit_rows(l1_v, 0, pair_v, 0, 8, 256)
        pltpu.sync_copy(
            pair_v, tbl.at[pl.ds(pbase + _OFFS[1] + slab * 2048, 2048), :])
        for j in range(4):
            _ds_pair(l1_v, (2 * j) * 256, (2 * j + 1) * 256, l2_v, j * 128, 128)
        _emit_rows(l2_v, 0, pair_v, 0, 4, 128)
        pltpu.sync_copy(
            pair_v.at[pl.ds(0, 512), :],
            tbl.at[pl.ds(pbase + _OFFS[2] + slab * 512, 512), :])
        for j in range(2):
            _ds_pair(l2_v, (2 * j) * 128, (2 * j + 1) * 128, l3_v, j * 64, 64)
        _emit_rows(l3_v, 0, pair_v, 0, 2, 64)
        pltpu.sync_copy(
            pair_v.at[pl.ds(0, 128), :],
            tbl.at[pl.ds(pbase + _OFFS[3] + slab * 128, 128), :])
        _ds_pair(l3_v, 0, 64, l4_v, 0, 32)
        _emit_rows(l4_v, 0, pair_v, 0, 1, 32)
        pltpu.sync_copy(
            pair_v.at[pl.ds(0, 32), :],
            tbl.at[pl.ds(pbase + _OFFS[4] + slab * 32, 32), :])
        pltpu.sync_copy(l4_v, l4f.at[plane, pl.ds(slab * 32, 32), :])


def _pyr14_kernel():
    return pl.kernel(
        _pyr14_body,
        out_type=jax.ShapeDtypeStruct((3, 1024, F), F32),
        mesh=_mesh(),
        compiler_params=_cparams(),
        scratch_types=[pltpu.VMEM((2048, F), F32),
                       pltpu.VMEM((2048, F), F32),
                       pltpu.VMEM((512, F), F32),
                       pltpu.VMEM((128, F), F32),
                       pltpu.VMEM((32, F), F32),
                       pltpu.VMEM((2048, F), I32)],
    )


def _pyr57_body(l4f, tbl, in_v, l5_v, l6_v, l7_v, pair_v):
    w = _wid()

    @pl.when(w < 3)
    def _():
        pltpu.sync_copy(l4f.at[w, :, :], in_v)

        def b5(r, _):
            _ds_pair(in_v, (2 * r) * 32, (2 * r + 1) * 32, l5_v, r * 16, 16)
            return 0
        lax.fori_loop(0, 16, b5, 0)
        _emit_rows(l5_v, 0, pair_v, 0, 16, 16)
        pltpu.sync_copy(pair_v.at[pl.ds(0, 256), :],
                        tbl.at[pl.ds(w * TPP + _OFFS[5], 256), :])

        def b6(r, _):
            _ds_pair(l5_v, (2 * r) * 16, (2 * r + 1) * 16, l6_v, r * 8, 8)
            return 0
        lax.fori_loop(0, 8, b6, 0)
        _emit_rows(l6_v, 0, pair_v, 0, 8, 8)
        pltpu.sync_copy(pair_v.at[pl.ds(0, 64), :],
                        tbl.at[pl.ds(w * TPP + _OFFS[6], 64), :])

        def b7(r, _):
            _ds_pair(l6_v, (2 * r) * 8, (2 * r + 1) * 8, l7_v, r * 4, 4)
            return 0
        lax.fori_loop(0, 4, b7, 0)
        _emit_rows(l7_v, 0, pair_v, 0, 4, 4)
        pltpu.sync_copy(pair_v.at[pl.ds(0, 16), :],
                        tbl.at[pl.ds(w * TPP + _OFFS[7], 16), :])


def _pyr57_kernel():
    return pl.kernel(
        _pyr57_body,
        out_type=(),
        mesh=_mesh(),
        compiler_params=_cparams(),
        scratch_types=[pltpu.VMEM((1024, F), F32),
                       pltpu.VMEM((256, F), F32),
                       pltpu.VMEM((64, F), F32),
                       pltpu.VMEM((16, F), F32),
                       pltpu.VMEM((256, F), I32)],
    )


_BN = 8192


def _uvl_body(x_ref, l_ref, x0_ref, x1_ref, x2_ref, lv_ref):
    xb = x_ref[...]
    xt = jnp.transpose(xb, (1, 0))
    x0_ref[...] = xt[0].reshape(_BN // 128, 128)
    x1_ref[...] = xt[1].reshape(_BN // 128, 128)
    x2_ref[...] = xt[2].reshape(_BN // 128, 128)
    lv_ref[...] = l_ref[...].reshape(_BN // 128, 128)


def _uvl(x, level):
    n_lin = NPTS // 128
    bspec = pl.BlockSpec((_BN // 128, 128), lambda i: (i, 0))
    return pl.pallas_call(
        _uvl_body,
        grid=(NPTS // _BN,),
        in_specs=[pl.BlockSpec((_BN, 3), lambda i: (i, 0)),
                  pl.BlockSpec((_BN, 1), lambda i: (i, 0))],
        out_specs=[bspec, bspec, bspec, bspec],
        out_shape=[jax.ShapeDtypeStruct((n_lin, 128), F32)] * 4,
    )(x, level)


_BC = 512
_NCH = 3 * (NPTS // NW) // _BC


def _main_body(tbl, x0_hbm, x1_hbm, x2_hbm, lev_hbm, out_hbm,
               u_v0, u_v1, v_v0, v_v1, lev_v0, lev_v1, idx_v0, idx_v1,
               w8_v0, w8_v1, rows_v0, rows_v1, outst_v, semg0, semg1):
    w = _wid()
    npw = NPTS // NW
    cpp = npw // _BC
    iota = _iota16()

    def chunk_coords(t):
        p = t // cpp
        c = t - p * cpp
        return p, w * npw + c * _BC

    def load_and_fire(t, u_v, v_v, lev_v, idx_v, w8_v, rows_v, semg):
        p, n0 = chunk_coords(t)
        r0 = n0 // 128
        nr = _BC // 128

        @pl.when(p == 0)
        def _():
            pltpu.sync_copy(x1_hbm.at[pl.ds(r0, nr), :], u_v)
            pltpu.sync_copy(x2_hbm.at[pl.ds(r0, nr), :], v_v)

        @pl.when(p == 1)
        def _():
            pltpu.sync_copy(x0_hbm.at[pl.ds(r0, nr), :], u_v)
            pltpu.sync_copy(x2_hbm.at[pl.ds(r0, nr), :], v_v)

        @pl.when(p == 2)
        def _():
            pltpu.sync_copy(x0_hbm.at[pl.ds(r0, nr), :], u_v)
            pltpu.sync_copy(x1_hbm.at[pl.ds(r0, nr), :], v_v)
        pltpu.sync_copy(lev_hbm.at[pl.ds(r0, nr), :], lev_v)
        prow = p * TPP

        def grp(g, _):
            gr = g >> 3
            gc = (g & 7) * 16
            u = u_v[gr, pl.ds(gc, 16)]
            v = v_v[gr, pl.ds(gc, 16)]
            lv = lev_v[gr, pl.ds(gc, 16)]
            lv = jnp.clip(lv, 0.0, float(NLEV - 1))
            l0i = jnp.clip(lv.astype(I32), 0, NLEV - 1)
            l1i = jnp.minimum(l0i + 1, NLEV - 1)
            fl = lv - l0i.astype(F32)
            for s, (li, wl) in enumerate(((l0i, 1.0 - fl), (l1i, fl))):
                resi = lax.shift_right_logical(jnp.full((16,), R0, I32), li)
                resf = resi.astype(F32)
                offs = jnp.full((16,), _OFFS[NLEV - 1], I32)
                for l in range(NLEV - 2, -1, -1):
                    offs = jnp.where(li == l, _OFFS[l], offs)
                uu = u * resf - 0.5
                vv = v * resf - 0.5
                x0i = (uu + 1.0).astype(I32) - 1
                y0i = (vv + 1.0).astype(I32) - 1
                px = jnp.clip(x0i, 0, resi - 2)
                fxp = jnp.clip(uu - px.astype(F32), 0.0, 1.0)
                iy0 = jnp.clip(y0i, 0, resi - 1)
                iy1 = jnp.clip(y0i + 1, 0, resi - 1)
                fy = vv - y0i.astype(F32)
                base = prow + offs + px
                for jy, (iy, wy) in enumerate(((iy0, 1.0 - fy), (iy1, fy))):
                    k2 = s * 2 + jy
                    idx_v[k2, pl.ds(g * 16, 16)] = base + iy * resi
                    wly = wl * wy
                    w8_v[2 * k2, pl.ds(g * 16, 16)] = wly * (1.0 - fxp)
                    w8_v[2 * k2 + 1, pl.ds(g * 16, 16)] = wly * fxp
            return 0
        lax.fori_loop(0, _BC // 16, grp, 0)
        for k2 in range(4):
            for q in range(_BC // 128):
                pltpu.async_copy(
                    tbl.at[idx_v.at[k2, pl.ds(q * 128, 128)]],
                    rows_v.at[pl.ds((k2 * (_BC // 128) + q) * 128, 128), :],
                    semg)

    def drain(rows_v, semg):
        pltpu.make_async_copy(
            tbl.at[pl.ds(0, 4 * _BC), :], rows_v, semg).wait()

    def compute_store(t, w8_v, rows_v):
        p, n0 = chunk_coords(t)
        mhi = jnp.full((16,), -65536, I32)

        def grp(g, _):
            p0 = g * 16
            wvec = [w8_v[k, pl.ds(p0, 16)] for k in range(8)]
            for i in range(16):
                lane = jnp.full((16,), i, I32)
                acc = None
                for k2 in range(4):
                    wrd = rows_v[k2 * _BC + p0 + i]
                    tl = plsc.bitcast(lax.shift_left(wrd, 16), F32)
                    tr = plsc.bitcast(lax.bitwise_and(wrd, mhi), F32)
                    a = wvec[2 * k2][lane] * tl + wvec[2 * k2 + 1][lane] * tr
                    acc = a if acc is None else acc + a
                outst_v[p0 + i] = acc
            return 0
        lax.fori_loop(0, _BC // 16, grp, 0)
        pltpu.sync_copy(outst_v,
                        out_hbm.at[pl.ds(n0, _BC), pl.ds(p * F, F)])

    set0 = (u_v0, v_v0, lev_v0, idx_v0, w8_v0, rows_v0, semg0)
    set1 = (u_v1, v_v1, lev_v1, idx_v1, w8_v1, rows_v1, semg1)
    load_and_fire(0, *set0)

    def body(i, _):
        t0 = 2 * i
        load_and_fire(t0 + 1, *set1)
        drain(rows_v0, semg0)
        compute_store(t0, w8_v0, rows_v0)

        @pl.when(t0 + 2 < _NCH)
        def _():
            load_and_fire(t0 + 2, *set0)
        drain(rows_v1, semg1)
        compute_store(t0 + 1, w8_v1, rows_v1)
        return 0
    lax.fori_loop(0, _NCH // 2, body, 0)


def _main_kernel():
    return pl.kernel(
        _main_body,
        out_type=jax.ShapeDtypeStruct((NPTS, 3 * F), F32),
        mesh=_mesh(),
        compiler_params=_cparams(),
        scratch_types=[pltpu.VMEM((_BC // 128, 128), F32),
                       pltpu.VMEM((_BC // 128, 128), F32),
                       pltpu.VMEM((_BC // 128, 128), F32),
                       pltpu.VMEM((_BC // 128, 128), F32),
                       pltpu.VMEM((_BC // 128, 128), F32),
                       pltpu.VMEM((_BC // 128, 128), F32),
                       pltpu.VMEM((4, _BC), I32),
                       pltpu.VMEM((4, _BC), I32),
                       pltpu.VMEM((8, _BC), F32),
                       pltpu.VMEM((8, _BC), F32),
                       pltpu.VMEM((4 * _BC, F), I32),
                       pltpu.VMEM((4 * _BC, F), I32),
                       pltpu.VMEM((_BC, F), F32),
                       pltpu.SemaphoreType.DMA,
                       pltpu.SemaphoreType.DMA],
    )


def kernel(x, level, texture):
    if x.shape[0] == 0:
        return jnp.zeros([0, F * 3], dtype=F32)
    tbl_ref = jax.new_ref(jnp.zeros((3 * TPP, F), I32))
    l4f = _pyr14_kernel()(texture, tbl_ref)
    _pyr57_kernel()(l4f, tbl_ref)
    x0, x1, x2, lv = _uvl(x, level)
    return _main_kernel()(tbl_ref, x0, x1, x2, lv)


# speedup vs baseline: 1.3177x; 1.0203x over previous
import jax
import jax.numpy as jnp
from jax import lax
from jax.experimental import pallas as pl
from jax.experimental.pallas import tpu as pltpu
from jax.experimental.pallas import tpu_sc as plsc

F32 = jnp.float32
I32 = jnp.int32

NLEV = 8
R0 = 512
F = 16
NPTS = 262144
TPP = 349520
NC, NS = 2, 16
NW = NC * NS
_OFFS = [0, 262144, 327680, 344064, 348160, 349184, 349440, 349504]


def _mesh():
    return plsc.VectorSubcoreMesh(
        core_axis_name="c", subcore_axis_name="s",
        num_cores=NC, num_subcores=NS)


def _cparams():
    return pltpu.CompilerParams(
        use_tc_tiling_on_sc=False, needs_layout_passes=False)


def _wid():
    return lax.axis_index("s") * NC + lax.axis_index("c")


def _iota16():
    return lax.iota(I32, 16)


def _ds_pair(src, s0, s1, dst, d0, width_out):
    def body(ox, _):
        a = src[s0 + 2 * ox]
        b = src[s0 + 2 * ox + 1]
        c = src[s1 + 2 * ox]
        d = src[s1 + 2 * ox + 1]
        dst[d0 + ox] = (a + b + c + d) * 0.25
        return 0
    lax.fori_loop(0, width_out, body, 0)


def _pack_pair(a, b):
    ai = plsc.bitcast(a, I32) + 0x8000
    bi = plsc.bitcast(b, I32) + 0x8000
    lo = lax.bitwise_and(lax.shift_right_logical(ai, 16),
                         jnp.full((16,), 0xFFFF, I32))
    hi = lax.bitwise_and(bi, jnp.full((16,), -65536, I32))
    return lax.bitwise_or(lo, hi)


def _emit_rows(src, s0, dst, d0, nrows, width):
    def body(j, _):
        r = j // width
        xx = j - r * width
        a = src[s0 + j]
        b = src[s0 + r * width + jnp.minimum(xx + 1, width - 1)]
        dst[d0 + j] = _pack_pair(a, b)
        return 0
    lax.fori_loop(0, nrows * width, body, 0)


def _pyr14_body(tex, tbl, l4f, in_v, l1_v, l2_v, l3_v, l4_v, pair_v):
    slab = _wid()
    for plane in range(3):
        pbase = plane * TPP
        for i in range(4):
            for r in range(4):
                pltpu.sync_copy(
                    tex.at[plane, slab * 16 + i * 4 + r, :, :],
                    in_v.at[pl.ds(r * 512, 512), :])
            _emit_rows(in_v, 0, pair_v, 0, 4, 512)
            pltpu.sync_copy(
                pair_v,
                tbl.at[pl.ds(pbase + slab * 8192 + i * 2048, 2048), :])
            for jj in range(2):
                _ds_pair(in_v, jj * 1024, jj * 1024 + 512,
                         l1_v, (i * 2 + jj) * 256, 256)
        _emit_rows(l1_v, 0, pair_v, 0, 8, 256)
        pltpu.sync_copy(
            pair_v, tbl.at[pl.ds(pbase + _OFFS[1] + slab * 2048, 2048), :])
        for j in range(4):
            _ds_pair(l1_v, (2 * j) * 256, (2 * j + 1) * 256, l2_v, j * 128, 128)
        _emit_rows(l2_v, 0, pair_v, 0, 4, 128)
        pltpu.sync_copy(
            pair_v.at[pl.ds(0, 512), :],
            tbl.at[pl.ds(pbase + _OFFS[2] + slab * 512, 512), :])
        for j in range(2):
            _ds_pair(l2_v, (2 * j) * 128, (2 * j + 1) * 128, l3_v, j * 64, 64)
        _emit_rows(l3_v, 0, pair_v, 0, 2, 64)
        pltpu.sync_copy(
            pair_v.at[pl.ds(0, 128), :],
            tbl.at[pl.ds(pbase + _OFFS[3] + slab * 128, 128), :])
        _ds_pair(l3_v, 0, 64, l4_v, 0, 32)
        _emit_rows(l4_v, 0, pair_v, 0, 1, 32)
        pltpu.sync_copy(
            pair_v.at[pl.ds(0, 32), :],
            tbl.at[pl.ds(pbase + _OFFS[4] + slab * 32, 32), :])
        pltpu.sync_copy(l4_v, l4f.at[plane, pl.ds(slab * 32, 32), :])


def _pyr14_kernel():
    return pl.kernel(
        _pyr14_body,
        out_type=jax.ShapeDtypeStruct((3, 1024, F), F32),
        mesh=_mesh(),
        compiler_params=_cparams(),
        scratch_types=[pltpu.VMEM((2048, F), F32),
                       pltpu.VMEM((2048, F), F32),
                       pltpu.VMEM((512, F), F32),
                       pltpu.VMEM((128, F), F32),
                       pltpu.VMEM((32, F), F32),
                       pltpu.VMEM((2048, F), I32)],
    )


def _pyr57_body(l4f, tbl, in_v, l5_v, l6_v, l7_v, pair_v):
    w = _wid()

    @pl.when(w < 3)
    def _():
        pltpu.sync_copy(l4f.at[w, :, :], in_v)

        def b5(r, _):
            _ds_pair(in_v, (2 * r) * 32, (2 * r + 1) * 32, l5_v, r * 16, 16)
            return 0
        lax.fori_loop(0, 16, b5, 0)
        _emit_rows(l5_v, 0, pair_v, 0, 16, 16)
        pltpu.sync_copy(pair_v.at[pl.ds(0, 256), :],
                        tbl.at[pl.ds(w * TPP + _OFFS[5], 256), :])

        def b6(r, _):
            _ds_pair(l5_v, (2 * r) * 16, (2 * r + 1) * 16, l6_v, r * 8, 8)
            return 0
        lax.fori_loop(0, 8, b6, 0)
        _emit_rows(l6_v, 0, pair_v, 0, 8, 8)
        pltpu.sync_copy(pair_v.at[pl.ds(0, 64), :],
                        tbl.at[pl.ds(w * TPP + _OFFS[6], 64), :])

        def b7(r, _):
            _ds_pair(l6_v, (2 * r) * 8, (2 * r + 1) * 8, l7_v, r * 4, 4)
            return 0
        lax.fori_loop(0, 4, b7, 0)
        _emit_rows(l7_v, 0, pair_v, 0, 4, 4)
        pltpu.sync_copy(pair_v.at[pl.ds(0, 16), :],
                        tbl.at[pl.ds(w * TPP + _OFFS[7], 16), :])


def _pyr57_kernel():
    return pl.kernel(
        _pyr57_body,
        out_type=(),
        mesh=_mesh(),
        compiler_params=_cparams(),
        scratch_types=[pltpu.VMEM((1024, F), F32),
                       pltpu.VMEM((256, F), F32),
                       pltpu.VMEM((64, F), F32),
                       pltpu.VMEM((16, F), F32),
                       pltpu.VMEM((256, F), I32)],
    )


_BN = 8192


def _uvl_body(x_ref, l_ref, x0_ref, x1_ref, x2_ref, lv_ref):
    xb = x_ref[...]
    xt = jnp.transpose(xb, (1, 0))
    x0_ref[...] = xt[0].reshape(_BN // 128, 128)
    x1_ref[...] = xt[1].reshape(_BN // 128, 128)
    x2_ref[...] = xt[2].reshape(_BN // 128, 128)
    lv_ref[...] = l_ref[...].reshape(_BN // 128, 128)


def _uvl(x, level):
    n_lin = NPTS // 128
    bspec = pl.BlockSpec((_BN // 128, 128), lambda i: (i, 0))
    return pl.pallas_call(
        _uvl_body,
        grid=(NPTS // _BN,),
        in_specs=[pl.BlockSpec((_BN, 3), lambda i: (i, 0)),
                  pl.BlockSpec((_BN, 1), lambda i: (i, 0))],
        out_specs=[bspec, bspec, bspec, bspec],
        out_shape=[jax.ShapeDtypeStruct((n_lin, 128), F32)] * 4,
    )(x, level)


_BC = 512
_NCH = 3 * (NPTS // NW) // _BC


def _main_body(tbl, x0_hbm, x1_hbm, x2_hbm, lev_hbm, out_hbm,
               u_v0, u_v1, v_v0, v_v1, lev_v0, lev_v1, idx_v0, idx_v1,
               w8_v0, w8_v1, rows_v0, rows_v1, outst_v, semg0, semg1):
    w = _wid()
    npw = NPTS // NW
    cpp = npw // _BC
    iota = _iota16()

    def chunk_coords(t):
        p = t // cpp
        c = t - p * cpp
        return p, w * npw + c * _BC

    def load_and_fire(t, u_v, v_v, lev_v, idx_v, w8_v, rows_v, semg):
        p, n0 = chunk_coords(t)
        r0 = n0 // 128
        nr = _BC // 128

        @pl.when(p == 0)
        def _():
            pltpu.sync_copy(x1_hbm.at[pl.ds(r0, nr), :], u_v)
            pltpu.sync_copy(x2_hbm.at[pl.ds(r0, nr), :], v_v)

        @pl.when(p == 1)
        def _():
            pltpu.sync_copy(x0_hbm.at[pl.ds(r0, nr), :], u_v)
            pltpu.sync_copy(x2_hbm.at[pl.ds(r0, nr), :], v_v)

        @pl.when(p == 2)
        def _():
            pltpu.sync_copy(x0_hbm.at[pl.ds(r0, nr), :], u_v)
            pltpu.sync_copy(x1_hbm.at[pl.ds(r0, nr), :], v_v)
        pltpu.sync_copy(lev_hbm.at[pl.ds(r0, nr), :], lev_v)
        prow = p * TPP

        def grp(g, _):
            gr = g >> 3
            gc = (g & 7) * 16
            u = u_v[gr, pl.ds(gc, 16)]
            v = v_v[gr, pl.ds(gc, 16)]
            lv = lev_v[gr, pl.ds(gc, 16)]
            lv = jnp.clip(lv, 0.0, float(NLEV - 1))
            l0i = jnp.clip(lv.astype(I32), 0, NLEV - 1)
            l1i = jnp.minimum(l0i + 1, NLEV - 1)
            fl = lv - l0i.astype(F32)
            for s, (li, wl) in enumerate(((l0i, 1.0 - fl), (l1i, fl))):
                resi = lax.shift_right_logical(jnp.full((16,), R0, I32), li)
                resf = resi.astype(F32)
                offs = jnp.full((16,), _OFFS[NLEV - 1], I32)
                for l in range(NLEV - 2, -1, -1):
                    offs = jnp.where(li == l, _OFFS[l], offs)
                uu = u * resf - 0.5
                vv = v * resf - 0.5
                x0i = (uu + 1.0).astype(I32) - 1
                y0i = (vv + 1.0).astype(I32) - 1
                px = jnp.clip(x0i, 0, resi - 2)
                fxp = jnp.clip(uu - px.astype(F32), 0.0, 1.0)
                iy0 = jnp.clip(y0i, 0, resi - 1)
                iy1 = jnp.clip(y0i + 1, 0, resi - 1)
                fy = vv - y0i.astype(F32)
                base = prow + offs + px
                for jy, (iy, wy) in enumerate(((iy0, 1.0 - fy), (iy1, fy))):
                    k2 = s * 2 + jy
                    idx_v[k2, pl.ds(g * 16, 16)] = base + iy * resi
                    wly = wl * wy
                    w8_v[2 * k2, pl.ds(g * 16, 16)] = wly * (1.0 - fxp)
                    w8_v[2 * k2 + 1, pl.ds(g * 16, 16)] = wly * fxp
            return 0
        lax.fori_loop(0, _BC // 16, grp, 0)
        for k2 in range(4):
            for q in range(_BC // 128):
                pltpu.async_copy(
                    tbl.at[idx_v.at[k2, pl.ds(q * 128, 128)]],
                    rows_v.at[pl.ds((k2 * (_BC // 128) + q) * 128, 128), :],
                    semg)

    def drain(rows_v, semg):
        pltpu.make_async_copy(
            tbl.at[pl.ds(0, 4 * _BC), :], rows_v, semg).wait()

    def compute_store(t, w8_v, rows_v):
        p, n0 = chunk_coords(t)
        mhi = jnp.full((16,), -65536, I32)

        def grp(g, _):
            p0 = g * 16
            wvec = [w8_v[k, pl.ds(p0, 16)] for k in range(8)]
            for i in range(16):
                lane = jnp.full((16,), i, I32)
                acc = None
                for k2 in range(4):
                    wrd = rows_v[k2 * _BC + p0 + i]
                    tl = plsc.bitcast(lax.shift_left(wrd, 16), F32)
                    tr = plsc.bitcast(lax.bitwise_and(wrd, mhi), F32)
                    a = wvec[2 * k2][lane] * tl + wvec[2 * k2 + 1][lane] * tr
                    acc = a if acc is None else acc + a
                outst_v[p0 + i] = acc
            return 0
        lax.fori_loop(0, _BC // 16, grp, 0)
        pltpu.sync_copy(outst_v,
                        out_hbm.at[pl.ds(n0, _BC), pl.ds(p * F, F)])

    set0 = (u_v0, v_v0, lev_v0, idx_v0, w8_v0, rows_v0, semg0)
    set1 = (u_v1, v_v1, lev_v1, idx_v1, w8_v1, rows_v1, semg1)
    load_and_fire(0, *set0)

    def body(i, _):
        t0 = 2 * i
        load_and_fire(t0 + 1, *set1)
        drain(rows_v0, semg0)
        compute_store(t0, w8_v0, rows_v0)

        @pl.when(t0 + 2 < _NCH)
        def _():
            load_and_fire(t0 + 2, *set0)
        drain(rows_v1, semg1)
        compute_store(t0 + 1, w8_v1, rows_v1)
        return 0
    lax.fori_loop(0, _NCH // 2, body, 0)


def _main_kernel():
    return pl.kernel(
        _main_body,
        out_type=jax.ShapeDtypeStruct((NPTS, 3 * F), F32),
        mesh=_mesh(),
        compiler_params=_cparams(),
        scratch_types=[pltpu.VMEM((_BC // 128, 128), F32),
                       pltpu.VMEM((_BC // 128, 128), F32),
                       pltpu.VMEM((_BC // 128, 128), F32),
                       pltpu.VMEM((_BC // 128, 128), F32),
                       pltpu.VMEM((_BC // 128, 128), F32),
                       pltpu.VMEM((_BC // 128, 128), F32),
                       pltpu.VMEM((4, _BC), I32),
                       pltpu.VMEM((4, _BC), I32),
                       pltpu.VMEM((8, _BC), F32),
                       pltpu.VMEM((8, _BC), F32),
                       pltpu.VMEM((4 * _BC, F), I32),
                       pltpu.VMEM((4 * _BC, F), I32),
                       pltpu.VMEM((_BC, F), F32),
                       pltpu.SemaphoreType.DMA,
                       pltpu.SemaphoreType.DMA],
    )


def kernel(x, level, texture):
    if x.shape[0] == 0:
        return jnp.zeros([0, F * 3], dtype=F32)
    tbl_ref = jax.new_ref(jnp.zeros((3 * TPP, F), I32))
    l4f = _pyr14_kernel()(texture, tbl_ref)
    _pyr57_kernel()(l4f, tbl_ref)
    x0, x1, x2, lv = _uvl(x, level)
    return _main_kernel()(jax.freeze(tbl_ref), x0, x1, x2, lv)
